# baseline (speedup 1.0000x reference)
import functools

import numpy as np
import jax
import jax.numpy as jnp
from jax import lax
from jax.experimental import pallas as pl
from jax.experimental.pallas import tpu as pltpu
from jax.experimental.pallas import tpu_sc as plsc

_CHUNK = 128
_BLK = 2000
_NS = 16
_NC = 2


def _sc_aggregate(xcat, ei32, N, E, half):
    nchunk = E // _CHUNK
    maxk = -(-nchunk // _NS)
    rpt = 640
    last_rpt = N - (_NS - 1) * rpt

    mesh = plsc.VectorSubcoreMesh(core_axis_name="c", subcore_axis_name="s")

    @functools.partial(
        pl.kernel,
        mesh=mesh,
        compiler_params=pltpu.CompilerParams(needs_layout_passes=False),
        out_type=[
            jax.ShapeDtypeStruct((_NC, N, half), jnp.float32),
            jax.ShapeDtypeStruct((N // _BLK, _NC * _NS, 1, _BLK),
                                 jnp.float32),
        ],
        scratch_types=[
            pltpu.VMEM((2, _CHUNK), jnp.int32),
            pltpu.VMEM((2, _CHUNK), jnp.int32),
            pltpu.VMEM((2, _CHUNK), jnp.int32),
            pltpu.VMEM((2, _CHUNK), jnp.int32),
            pltpu.VMEM((_CHUNK, half), jnp.float32),
            pltpu.VMEM((_CHUNK, half), jnp.float32),
            pltpu.VMEM((N,), jnp.float32),
            pltpu.VMEM_SHARED((N, half), jnp.float32),
            pltpu.SemaphoreType.DMA,
            pltpu.SemaphoreType.DMA,
            pltpu.SemaphoreType.DMA,
            pltpu.SemaphoreType.DMA,
            pltpu.SemaphoreType.DMA,
            pltpu.SemaphoreType.DMA,
        ],
    )
    def sc_kernel(xcat_h, ei_h, agg_h, deg_h,
                  idx0, idx1, idx2, idx3, rows0, rows1, degp, acc,
                  semg0, semg1, semi0, semi1, semi2, semi3):
        c = lax.axis_index("c")
        w = lax.axis_index("s")
        z16 = jnp.zeros((16,), jnp.float32)

        def zfill(i, carry):
            for j in range(half // 16):
                rows0[i, pl.ds(j * 16, 16)] = z16
            return carry

        lax.fori_loop(jnp.int32(0), jnp.int32(_CHUNK), zfill, jnp.int32(0))

        @pl.when(w < _NS - 1)
        def _():
            for s in range(rpt // _CHUNK):
                pltpu.sync_copy(rows0,
                                acc.at[pl.ds(w * rpt + s * _CHUNK, _CHUNK)])

        @pl.when(w == _NS - 1)
        def _():
            for s in range(last_rpt // _CHUNK):
                pltpu.sync_copy(
                    rows0,
                    acc.at[pl.ds((_NS - 1) * rpt + s * _CHUNK, _CHUNK)])
            rem = last_rpt % _CHUNK
            if rem:
                pltpu.sync_copy(
                    rows0.at[pl.ds(0, rem)],
                    acc.at[pl.ds((_NS - 1) * rpt + last_rpt - rem, rem)])

        def zdeg(i, carry):
            degp[pl.ds(pl.multiple_of(i * 16, 16), 16)] = z16
            return carry

        lax.fori_loop(jnp.int32(0), jnp.int32(N // 16), zdeg, jnp.int32(0))

        plsc.subcore_barrier()

        ones16 = jnp.full((16,), 1.0, dtype=jnp.float32)

        ibufs = [(idx0, semi0), (idx1, semi1), (idx2, semi2), (idx3, semi3)]

        def load_idx(k, r):
            cid = w + _NS * k
            idxb, semi = ibufs[r]

            @pl.when(cid < nchunk)
            def _():
                base = pl.multiple_of(cid * _CHUNK, _CHUNK)
                pltpu.async_copy(ei_h.at[:, pl.ds(base, _CHUNK)],
                                 idxb, semi)

        def issue_gather(k, r, rowsb, semg):
            cid = w + _NS * k
            idxb, semi = ibufs[r]

            @pl.when(cid < nchunk)
            def _():
                base = pl.multiple_of(cid * _CHUNK, _CHUNK)
                pltpu.make_async_copy(ei_h.at[:, pl.ds(base, _CHUNK)],
                                      idxb, semi).wait()
                for j in range(_CHUNK // 16):
                    sl = pl.ds(j * 16, 16)
                    idxb[0, sl] = idxb[0, sl] * 2 + c
                pltpu.async_copy(xcat_h.at[idxb.at[jnp.int32(0)]],
                                 rowsb, semg)

            load_idx(k + 2, (r + 2) % 4)

        def finish(k, r, rowsb, semg):
            cid = w + _NS * k
            idxb, _ = ibufs[r]

            @pl.when(cid < nchunk)
            def _():
                pltpu.make_async_copy(xcat_h.at[idxb.at[jnp.int32(0)]],
                                      rowsb, semg).wait()
                pltpu.sync_copy(rowsb, acc.at[idxb.at[jnp.int32(1)]],
                                add=True)
                cbase = c * (_CHUNK // 2)
                for j in range(_CHUNK // 32):
                    plsc.addupdate_scatter(
                        degp, [idxb[1, pl.ds(cbase + j * 16, 16)]], ones16)

        load_idx(0, 0)
        load_idx(1, 1)
        issue_gather(0, 0, rows0, semg0)
        issue_gather(1, 1, rows1, semg1)

        def quad(i, carry):
            k0 = 4 * i
            for b in range(4):
                rb, sg = (rows0, semg0) if b % 2 == 0 else (rows1, semg1)
                finish(k0 + b, b, rb, sg)
                issue_gather(k0 + b + 2, (b + 2) % 4, rb, sg)
            return carry

        lax.fori_loop(jnp.int32(0), jnp.int32((maxk + 3) // 4), quad,
                      jnp.int32(0))
        plsc.subcore_barrier()

        @pl.when(w < _NS - 1)
        def _():
            pltpu.sync_copy(acc.at[pl.ds(w * rpt, rpt)],
                            agg_h.at[c, pl.ds(w * rpt, rpt)])

        @pl.when(w == _NS - 1)
        def _():
            pltpu.sync_copy(acc.at[pl.ds((_NS - 1) * rpt, last_rpt)],
                            agg_h.at[c, pl.ds((_NS - 1) * rpt, last_rpt)])

        for g in range(N // _BLK):
            pltpu.sync_copy(degp.at[pl.ds(g * _BLK, _BLK)],
                            deg_h.at[jnp.int32(g), c * _NS + w,
                                     jnp.int32(0)])

    return sc_kernel(xcat, ei32)


def _tc_fused(x, agg3, deg16, W1, b1, W2, b2, C, W3, b3, W4, b4):
    N, D = x.shape
    K, CD = C.shape
    half = D // 2
    G = N // _BLK
    f32 = jnp.float32

    def body(x_ref, a3_ref, dg_ref, W1_ref, b1_ref, W2_ref, b2_ref, C_ref,
             W3_ref, b3_ref, W4_ref, b4_ref, xr_ref, loss_ref, idx_ref):
        i = pl.program_id(0)
        xb = x_ref[...]
        deg = jnp.sum(dg_ref[0, :, 0, :], axis=0) + 1.0
        hlo = (a3_ref[0] + xb[:, :half]) / deg[:, None]
        hhi = (a3_ref[1] + xb[:, half:]) / deg[:, None]
        h1 = jnp.maximum(
            jnp.dot(hlo, W1_ref[:half, :], preferred_element_type=f32)
            + jnp.dot(hhi, W1_ref[half:, :], preferred_element_type=f32)
            + b1_ref[...][None, :], 0.0)
        z = (jnp.dot(h1, W2_ref[...], preferred_element_type=f32)
             + b2_ref[...][None, :])
        Cb = C_ref[...]
        csq = jnp.sum(Cb * Cb, axis=1)
        zsq = jnp.sum(z * z, axis=1, keepdims=True)
        zc = lax.dot_general(z, Cb, (((1,), (1,)), ((), ())),
                             preferred_element_type=f32)
        d2 = zsq - 2.0 * zc + csq[None, :]
        m = jnp.min(d2, axis=1, keepdims=True)
        iota = lax.broadcasted_iota(jnp.int32, d2.shape, 1)
        idx = jnp.min(jnp.where(d2 == m, iota, K), axis=1)
        oh = (iota == idx[:, None]).astype(f32)
        zq = jnp.dot(oh, Cb, preferred_element_type=f32)
        diff = zq - z
        lp = jnp.sum(diff * diff)

        @pl.when(i == 0)
        def _():
            loss_ref[...] = jnp.zeros_like(loss_ref)

        loss_ref[...] = loss_ref[...] + lp * (1.25 / (N * CD))
        h2 = jnp.maximum(
            jnp.dot(zq, W3_ref[...], preferred_element_type=f32)
            + b3_ref[...][None, :], 0.0)
        xr_ref[...] = (jnp.dot(h2, W4_ref[...], preferred_element_type=f32)
                       + b4_ref[...][None, :])
        idx_ref[...] = idx.reshape(1, 1, _BLK)

    z = np.int32(0)
    full2 = lambda shape: pl.BlockSpec(shape, lambda i: (z,) * len(shape))
    return pl.pallas_call(
        body,
        grid=(G,),
        in_specs=[
            pl.BlockSpec((_BLK, D), lambda i: (i, z)),
            pl.BlockSpec((_NC, _BLK, half), lambda i: (z, i, z)),
            pl.BlockSpec((1, _NC * _NS, 1, _BLK), lambda i: (i, z, z, z)),
            full2((D, D)), full2((D,)), full2((D, CD)), full2((CD,)),
            full2((K, CD)),
            full2((CD, D)), full2((D,)), full2((D, D)), full2((D,)),
        ],
        out_specs=[
            pl.BlockSpec((_BLK, D), lambda i: (i, z)),
            pl.BlockSpec((1, 1), lambda i: (z, z)),
            pl.BlockSpec((1, 1, _BLK), lambda i: (i, z, z)),
        ],
        out_shape=[
            jax.ShapeDtypeStruct((N, D), f32),
            jax.ShapeDtypeStruct((1, 1), f32),
            jax.ShapeDtypeStruct((G, 1, _BLK), jnp.int32),
        ],
    )(x, agg3, deg16, W1, b1, W2, b2, C, W3, b3, W4, b4)


def kernel(x, edge_index, W_enc1, b_enc1, W_enc2, b_enc2, codebook,
           W_dec1, b_dec1, W_dec2, b_dec2):
    N, D = x.shape
    E = edge_index.shape[1]
    half = D // 2

    ei32 = edge_index.astype(jnp.int32)
    xcat = x.reshape(2 * N, half)

    agg3, deg16 = _sc_aggregate(xcat, ei32, N, E, half)
    xrec, loss, idx3 = _tc_fused(x, agg3, deg16, W_enc1, b_enc1, W_enc2,
                                 b_enc2, codebook, W_dec1, b_dec1, W_dec2,
                                 b_dec2)
    indices = idx3.reshape(N).astype(jnp.int64)
    return xrec, loss[0, 0], indices
